# baseline (speedup 1.0000x reference)
import functools

import jax
import jax.numpy as jnp
from jax import lax
from jax.experimental import pallas as pl
from jax.experimental.pallas import tpu as pltpu
from jax.experimental.pallas import tpu_sc as plsc

_EMBED_DIM = 32
_HIST_PAD = 56
_LANE_PAD = 128
_NC = 2
_NS = 16
_NW = _NC * _NS
_NBUF = 4
_LOOKAHEAD = 2
_CHUNK = 800


def _sc_gather_to_img(table, idx32, batch, hist):
  n = batch * hist
  b_per_w = n // _NW
  rows_per_w = batch // _NW
  n_chunks = b_per_w // _CHUNK
  n_outer = n_chunks // _NBUF
  rows_per_chunk = _CHUNK // hist
  assert n_chunks % _NBUF == 0 and _CHUNK % hist == 0
  mesh = plsc.VectorSubcoreMesh(core_axis_name="c", subcore_axis_name="s")

  @functools.partial(
      pl.kernel,
      mesh=mesh,
      out_type=jax.ShapeDtypeStruct((batch, _HIST_PAD, _LANE_PAD), jnp.float32),
      compiler_params=pltpu.CompilerParams(use_tc_tiling_on_sc=False),
      scratch_types=[
          pltpu.VMEM((b_per_w,), jnp.int32),
          *[pltpu.VMEM((_CHUNK, _EMBED_DIM), jnp.float32) for _ in range(_NBUF)],
          *[pltpu.SemaphoreType.DMA for _ in range(2 * _NBUF)],
      ],
  )
  def k(table_hbm, idx_hbm, img_hbm, idx_v, *bufs_and_sems):
    rows = bufs_and_sems[:_NBUF]
    sg = bufs_and_sems[_NBUF:2 * _NBUF]
    sw = bufs_and_sems[2 * _NBUF:]
    wid = lax.axis_index("s") * _NC + lax.axis_index("c")
    row_base = wid * rows_per_w

    pltpu.sync_copy(idx_hbm.at[wid], idx_v)

    def gather_desc(g, b):
      return pltpu.make_async_copy(
          table_hbm.at[idx_v.at[pl.ds(g * _CHUNK, _CHUNK)]], rows[b], sg[b])

    def wb_descs(g, b):
      for j in range(rows_per_chunk):
        yield pltpu.make_async_copy(
            rows[b].at[pl.ds(j * hist, hist)],
            img_hbm.at[row_base + g * rows_per_chunk + j,
                       pl.ds(0, hist), pl.ds(0, _EMBED_DIM)],
            sw[b])

    for b in range(_LOOKAHEAD):
      gather_desc(b, b).start()

    def body(t, _):
      for b in range(_NBUF):
        g = t * _NBUF + b
        nxt = g + _LOOKAHEAD
        nb = (b + _LOOKAHEAD) % _NBUF
        prev = nxt - _NBUF

        @pl.when(jnp.logical_and(nxt < n_chunks, prev >= 0))
        def _():
          for d in wb_descs(prev, nb):
            d.wait()

        @pl.when(nxt < n_chunks)
        def _():
          gather_desc(nxt, nb).start()

        gather_desc(g, b).wait()
        for d in wb_descs(g, b):
          d.start()
      return ()

    lax.fori_loop(0, n_outer, body, ())

    for g in range(n_chunks - (_NBUF - _LOOKAHEAD), n_chunks):
      for d in wb_descs(g, g % _NBUF):
        d.wait()

  return k(table, idx32)


def _tc_depad(img, batch, hist):
  rb = 64
  grid = (batch // rb,)

  def body(img_ref, out_ref):
    out_ref[...] = img_ref[:, :hist, :_EMBED_DIM]

  return pl.pallas_call(
      body,
      grid=grid,
      in_specs=[pl.BlockSpec((rb, _HIST_PAD, _LANE_PAD), lambda i: (i, 0, 0))],
      out_specs=pl.BlockSpec((rb, hist, _EMBED_DIM), lambda i: (i, 0, 0)),
      out_shape=jax.ShapeDtypeStruct((batch, hist, _EMBED_DIM), jnp.float32),
  )(img)


def kernel(input_tokens, table):
  batch, hist = input_tokens.shape
  idx32 = input_tokens.reshape(_NW, (batch * hist) // _NW).astype(jnp.int32)
  img = _sc_gather_to_img(table, idx32, batch, hist)
  return _tc_depad(img, batch, hist)


# speedup vs baseline: 1.1700x; 1.1700x over previous
import functools

import jax
import jax.numpy as jnp
from jax import lax
from jax.experimental import pallas as pl
from jax.experimental.pallas import tpu as pltpu
from jax.experimental.pallas import tpu_sc as plsc

_EMBED_DIM = 32
_NC = 2
_NS = 16
_NW = _NC * _NS
_NBUF = 4
_LOOKAHEAD = 2
_CHUNK = 800


def _sc_gather(table, idx32, batch, hist):
  n = batch * hist
  b_per_w = n // _NW
  rows_per_w = batch // _NW
  n_chunks = b_per_w // _CHUNK
  n_outer = n_chunks // _NBUF
  rows_per_chunk = _CHUNK // hist
  assert n_chunks % _NBUF == 0 and _CHUNK % hist == 0
  mesh = plsc.VectorSubcoreMesh(core_axis_name="c", subcore_axis_name="s")

  @functools.partial(
      pl.kernel,
      mesh=mesh,
      out_type=jax.ShapeDtypeStruct((batch, hist, _EMBED_DIM), jnp.float32),
      compiler_params=pltpu.CompilerParams(use_tc_tiling_on_sc=False),
      scratch_types=[
          pltpu.VMEM((b_per_w,), jnp.int32),
          *[pltpu.VMEM((_CHUNK, _EMBED_DIM), jnp.float32) for _ in range(_NBUF)],
          *[pltpu.SemaphoreType.DMA for _ in range(2 * _NBUF)],
      ],
  )
  def k(table_hbm, idx_hbm, out_hbm, idx_v, *bufs_and_sems):
    rows = bufs_and_sems[:_NBUF]
    sg = bufs_and_sems[_NBUF:2 * _NBUF]
    sw = bufs_and_sems[2 * _NBUF:]
    wid = lax.axis_index("s") * _NC + lax.axis_index("c")
    row_base = wid * rows_per_w

    pltpu.sync_copy(idx_hbm.at[wid], idx_v)

    def gather_desc(g, b):
      return pltpu.make_async_copy(
          table_hbm.at[idx_v.at[pl.ds(g * _CHUNK, _CHUNK)]], rows[b], sg[b])

    def wb_descs(g, b):
      for j in range(rows_per_chunk):
        yield pltpu.make_async_copy(
            rows[b].at[pl.ds(j * hist, hist)],
            out_hbm.at[row_base + g * rows_per_chunk + j],
            sw[b])

    for b in range(_LOOKAHEAD):
      gather_desc(b, b).start()

    def body(t, _):
      for b in range(_NBUF):
        g = t * _NBUF + b
        nxt = g + _LOOKAHEAD
        nb = (b + _LOOKAHEAD) % _NBUF
        prev = nxt - _NBUF

        @pl.when(jnp.logical_and(nxt < n_chunks, prev >= 0))
        def _():
          for d in wb_descs(prev, nb):
            d.wait()

        @pl.when(nxt < n_chunks)
        def _():
          gather_desc(nxt, nb).start()

        gather_desc(g, b).wait()
        for d in wb_descs(g, b):
          d.start()
      return ()

    lax.fori_loop(0, n_outer, body, ())

    for g in range(n_chunks - (_NBUF - _LOOKAHEAD), n_chunks):
      for d in wb_descs(g, g % _NBUF):
        d.wait()

  return k(table, idx32)


def kernel(input_tokens, table):
  batch, hist = input_tokens.shape
  idx32 = input_tokens.reshape(_NW, (batch * hist) // _NW).astype(jnp.int32)
  return _sc_gather(table, idx32, batch, hist)
